# Initial kernel scaffold; baseline (speedup 1.0000x reference)
import functools

import jax
import jax.numpy as jnp
from jax import lax
from jax.experimental import pallas as pl
from jax.experimental.pallas import tpu as pltpu
from jax.experimental.pallas import tpu_sc as plsc

B, O, T, D, H, PO, P = 2, 10000, 160000, 128, 128, 128, 64
BT = B * T
W1B_OUT = 2 * H + PO
NC, NS = 2, 16
NW = NC * NS
EPW = BT // NW
CH = 80
NCH_G = EPW // CH
EPT = T // NS
NCH_S = EPT // CH
CW = D + 16
ZR = O // NS

_sc_mesh = plsc.VectorSubcoreMesh(core_axis_name="c", subcore_axis_name="s")


@functools.partial(
    pl.kernel,
    out_type=(jax.ShapeDtypeStruct((BT, D), jnp.float32),
              jax.ShapeDtypeStruct((BT, D), jnp.float32)),
    mesh=_sc_mesh,
    scratch_types=[
        pltpu.VMEM((CH,), jnp.int32),
        pltpu.VMEM((CH, D), jnp.float32),
        pltpu.VMEM((CH,), jnp.int32),
        pltpu.VMEM((CH, D), jnp.float32),
        pltpu.SemaphoreType.DMA,
        pltpu.SemaphoreType.DMA,
    ],
)
def _gather_k(obj_hbm, sidx_hbm, oidx_hbm, outs_hbm, outo_hbm,
              sidx_v, srows_v, oidx_v, orows_v, ssem, osem):
    wid = lax.axis_index("s") * NC + lax.axis_index("c")
    base = wid * EPW

    def chunk(i, carry):
        off = base + i * CH
        pltpu.sync_copy(sidx_hbm.at[pl.ds(off, CH)], sidx_v)
        pltpu.sync_copy(oidx_hbm.at[pl.ds(off, CH)], oidx_v)
        s_dma = pltpu.async_copy(obj_hbm.at[sidx_v], srows_v, ssem)
        o_dma = pltpu.async_copy(obj_hbm.at[oidx_v], orows_v, osem)
        s_dma.wait()
        o_dma.wait()
        pltpu.sync_copy(srows_v, outs_hbm.at[pl.ds(off, CH)])
        pltpu.sync_copy(orows_v, outo_hbm.at[pl.ds(off, CH)])
        return carry

    lax.fori_loop(0, NCH_G, chunk, 0)


@functools.partial(
    pl.kernel,
    out_type=jax.ShapeDtypeStruct((B * O, CW), jnp.float32),
    mesh=_sc_mesh,
    scratch_types=[
        pltpu.VMEM_SHARED((O, CW), jnp.float32),
        pltpu.VMEM((CH,), jnp.int32),
        pltpu.VMEM((CH, CW), jnp.float32),
        pltpu.VMEM((CH,), jnp.int32),
        pltpu.VMEM((CH, CW), jnp.float32),
    ],
)
def _scatter_k(sidx_hbm, oidx_hbm, cs_hbm, co_hbm, zeros_hbm, out_hbm,
               acc, sidx_v, srows_v, oidx_v, orows_v):
    c = lax.axis_index("c")
    sid = lax.axis_index("s")
    pltpu.sync_copy(zeros_hbm, acc.at[pl.ds(sid * ZR, ZR)])
    plsc.subcore_barrier()

    base = c * T + sid * EPT

    def chunk(i, carry):
        off = base + i * CH
        pltpu.sync_copy(sidx_hbm.at[pl.ds(off, CH)], sidx_v)
        pltpu.sync_copy(cs_hbm.at[pl.ds(off, CH)], srows_v)
        pltpu.sync_copy(srows_v, acc.at[sidx_v], add=True)
        pltpu.sync_copy(oidx_hbm.at[pl.ds(off, CH)], oidx_v)
        pltpu.sync_copy(co_hbm.at[pl.ds(off, CH)], orows_v)
        pltpu.sync_copy(orows_v, acc.at[oidx_v], add=True)
        return carry

    lax.fori_loop(0, NCH_S, chunk, 0)
    plsc.subcore_barrier()
    pltpu.sync_copy(acc.at[pl.ds(sid * ZR, ZR)],
                    out_hbm.at[pl.ds(c * O + sid * ZR, ZR)])


MLP_TILE = 512


def _mlp_body(s_ref, p_ref, o_ref, tt_ref, pid_ref, w_ref,
              w1s_ref, w1p_ref, w1o_ref, b1a_ref, w1b_ref, b1b_ref, ptw_ref,
              newp_ref, cs_ref, co_ref):
    s = s_ref[...]
    pv = p_ref[...]
    o = o_ref[...]
    h = (jnp.dot(s, w1s_ref[...], preferred_element_type=jnp.float32)
         + jnp.dot(pv, w1p_ref[...], preferred_element_type=jnp.float32)
         + jnp.dot(o, w1o_ref[...], preferred_element_type=jnp.float32)
         + b1a_ref[...])
    h = jnp.maximum(h, 0.0)
    new_t = jnp.dot(h, w1b_ref[...], preferred_element_type=jnp.float32) + b1b_ref[...]
    new_t = jnp.maximum(new_t, 0.0)

    ptp = jax.nn.sigmoid(ptw_ref[...])
    pid = pid_ref[...]
    lanes = lax.broadcasted_iota(jnp.float32, (MLP_TILE, P), 1)
    onehot = (lanes == pid).astype(jnp.float32)
    conf_t = jnp.sum(onehot * ptp, axis=1, keepdims=True)
    tt = tt_ref[...]
    conf = jnp.where(tt == 0.0, 1.0, conf_t)
    w = w_ref[...]
    cfw = conf * w

    newp_ref[...] = new_t[:, D:2 * D] * conf
    cs_ref[:, :D] = new_t[:, :D] * cfw
    cs_ref[:, D:CW] = jnp.broadcast_to(cfw, (MLP_TILE, 16))
    co_ref[:, :D] = new_t[:, 2 * D:] * cfw
    co_ref[:, D:CW] = jnp.broadcast_to(cfw, (MLP_TILE, 16))


_mlp_call = pl.pallas_call(
    _mlp_body,
    grid=(BT // MLP_TILE,),
    in_specs=[
        pl.BlockSpec((MLP_TILE, D), lambda i: (i, 0)),
        pl.BlockSpec((MLP_TILE, D), lambda i: (i, 0)),
        pl.BlockSpec((MLP_TILE, D), lambda i: (i, 0)),
        pl.BlockSpec((MLP_TILE, 1), lambda i: (i, 0)),
        pl.BlockSpec((MLP_TILE, 1), lambda i: (i, 0)),
        pl.BlockSpec((MLP_TILE, 1), lambda i: (i, 0)),
        pl.BlockSpec((D, H), lambda i: (0, 0)),
        pl.BlockSpec((D, H), lambda i: (0, 0)),
        pl.BlockSpec((D, H), lambda i: (0, 0)),
        pl.BlockSpec((1, H), lambda i: (0, 0)),
        pl.BlockSpec((H, W1B_OUT), lambda i: (0, 0)),
        pl.BlockSpec((1, W1B_OUT), lambda i: (0, 0)),
        pl.BlockSpec((1, P), lambda i: (0, 0)),
    ],
    out_specs=[
        pl.BlockSpec((MLP_TILE, D), lambda i: (i, 0)),
        pl.BlockSpec((MLP_TILE, CW), lambda i: (i, 0)),
        pl.BlockSpec((MLP_TILE, CW), lambda i: (i, 0)),
    ],
    out_shape=[
        jax.ShapeDtypeStruct((BT, D), jnp.float32),
        jax.ShapeDtypeStruct((BT, CW), jnp.float32),
        jax.ShapeDtypeStruct((BT, CW), jnp.float32),
    ],
    compiler_params=pltpu.CompilerParams(dimension_semantics=("arbitrary",)),
)


OUT_TILE = 2000


def _out_body(pp_ref, w2a_ref, b2a_ref, w2b_ref, b2b_ref, out_ref):
    x = pp_ref[...]
    pooled = x[:, :D]
    cnt = jnp.max(x[:, D:CW], axis=1, keepdims=True)
    denom = jnp.where(cnt > 0.0, cnt, 1.0)
    pn = pooled / denom
    h2 = jnp.maximum(
        jnp.dot(pn, w2a_ref[...], preferred_element_type=jnp.float32)
        + b2a_ref[...], 0.0)
    out_ref[...] = jnp.maximum(
        jnp.dot(h2, w2b_ref[...], preferred_element_type=jnp.float32)
        + b2b_ref[...], 0.0)


_out_call = pl.pallas_call(
    _out_body,
    grid=(B * O // OUT_TILE,),
    in_specs=[
        pl.BlockSpec((OUT_TILE, CW), lambda i: (i, 0)),
        pl.BlockSpec((H, H), lambda i: (0, 0)),
        pl.BlockSpec((1, H), lambda i: (0, 0)),
        pl.BlockSpec((H, D), lambda i: (0, 0)),
        pl.BlockSpec((1, D), lambda i: (0, 0)),
    ],
    out_specs=pl.BlockSpec((OUT_TILE, D), lambda i: (i, 0)),
    out_shape=jax.ShapeDtypeStruct((B * O, D), jnp.float32),
    compiler_params=pltpu.CompilerParams(dimension_semantics=("arbitrary",)),
)


def kernel(obj_vecs, pred_vecs, edges, pred_indicators, triplet_type,
           predicate_ids, W1a, b1a, W1b, b1b, W2a, b2a, W2b, b2b, ptw):
    s_idx = edges[:, :, 0]
    o_idx = edges[:, :, 1]
    boff = (jnp.arange(B, dtype=jnp.int32) * O)[:, None]
    sflat_g = (s_idx + boff).reshape(BT)
    oflat_g = (o_idx + boff).reshape(BT)
    obj_flat = obj_vecs.reshape(B * O, D)

    cur_s, cur_o = _gather_k(obj_flat, sflat_g, oflat_g)

    pred_flat = pred_vecs.reshape(BT, D)
    ttf = triplet_type.astype(jnp.float32).reshape(BT, 1)
    pidf = predicate_ids.astype(jnp.float32).reshape(BT, 1)
    wf = pred_indicators.astype(jnp.float32).reshape(BT, 1)

    new_p, cs, co = _mlp_call(
        cur_s, pred_flat, cur_o, ttf, pidf, wf,
        W1a[:D], W1a[D:2 * D], W1a[2 * D:], b1a.reshape(1, H),
        W1b, b1b.reshape(1, W1B_OUT), ptw.reshape(1, P))

    zeros = jnp.zeros((ZR, CW), jnp.float32)
    pp = _scatter_k(s_idx.reshape(BT), o_idx.reshape(BT), cs, co, zeros)

    new_obj = _out_call(pp, W2a, b2a.reshape(1, H), W2b, b2b.reshape(1, D))
    return new_obj.reshape(B, O, D), new_p.reshape(B, T, D)


# speedup vs baseline: 2006.5362x; 2006.5362x over previous
import functools

import jax
import jax.numpy as jnp
from jax import lax
from jax.experimental import pallas as pl
from jax.experimental.pallas import tpu as pltpu
from jax.experimental.pallas import tpu_sc as plsc

B, O, T, D, H, PO, P = 2, 10000, 160000, 128, 128, 128, 64
BT = B * T
W1B_OUT = 2 * H + PO
NC, NS = 2, 16
NW = NC * NS
EPW = BT // NW
CH = 80
NCH_G = EPW // CH
EPT = T // NS
NCH_S = EPT // CH
OPAD = 10240
ZR = OPAD // NS
CROWS = OPAD // D

_sc_mesh = plsc.VectorSubcoreMesh(core_axis_name="c", subcore_axis_name="s")


@functools.partial(
    pl.kernel,
    out_type=(jax.ShapeDtypeStruct((BT, D), jnp.float32),
              jax.ShapeDtypeStruct((BT, D), jnp.float32)),
    mesh=_sc_mesh,
    scratch_types=[
        pltpu.VMEM((CH,), jnp.int32),
        pltpu.VMEM((CH, D), jnp.float32),
        pltpu.VMEM((CH,), jnp.int32),
        pltpu.VMEM((CH, D), jnp.float32),
        pltpu.SemaphoreType.DMA,
        pltpu.SemaphoreType.DMA,
    ],
)
def _gather_k(obj_hbm, sidx_hbm, oidx_hbm, outs_hbm, outo_hbm,
              sidx_v, srows_v, oidx_v, orows_v, ssem, osem):
    wid = lax.axis_index("s") * NC + lax.axis_index("c")
    base = wid * EPW

    def chunk(i, carry):
        off = base + i * CH
        pltpu.sync_copy(sidx_hbm.at[pl.ds(off, CH)], sidx_v)
        pltpu.sync_copy(oidx_hbm.at[pl.ds(off, CH)], oidx_v)
        s_dma = pltpu.async_copy(obj_hbm.at[sidx_v], srows_v, ssem)
        o_dma = pltpu.async_copy(obj_hbm.at[oidx_v], orows_v, osem)
        s_dma.wait()
        o_dma.wait()
        pltpu.sync_copy(srows_v, outs_hbm.at[pl.ds(off, CH)])
        pltpu.sync_copy(orows_v, outo_hbm.at[pl.ds(off, CH)])
        return carry

    lax.fori_loop(0, NCH_G, chunk, 0)


@functools.partial(
    pl.kernel,
    out_type=jax.ShapeDtypeStruct((B * OPAD, D), jnp.float32),
    mesh=_sc_mesh,
    scratch_types=[
        pltpu.VMEM_SHARED((OPAD, D), jnp.float32),
        pltpu.VMEM((CH,), jnp.int32),
        pltpu.VMEM((CH, D), jnp.float32),
        pltpu.VMEM((CH,), jnp.int32),
        pltpu.VMEM((CH, D), jnp.float32),
    ],
)
def _scatter_k(sidx_hbm, oidx_hbm, cs_hbm, co_hbm, zeros_hbm, outv_hbm,
               acc, sidx_v, srows_v, oidx_v, orows_v):
    c = lax.axis_index("c")
    sid = lax.axis_index("s")
    pltpu.sync_copy(zeros_hbm.at[pl.ds(0, ZR)], acc.at[pl.ds(sid * ZR, ZR)])
    plsc.subcore_barrier()

    base = c * T + sid * EPT

    def chunk(i, carry):
        off = base + i * CH
        pltpu.sync_copy(sidx_hbm.at[pl.ds(off, CH)], sidx_v)
        pltpu.sync_copy(cs_hbm.at[pl.ds(off, CH)], srows_v)
        pltpu.sync_copy(srows_v, acc.at[sidx_v], add=True)
        pltpu.sync_copy(oidx_hbm.at[pl.ds(off, CH)], oidx_v)
        pltpu.sync_copy(co_hbm.at[pl.ds(off, CH)], orows_v)
        pltpu.sync_copy(orows_v, acc.at[oidx_v], add=True)
        return carry

    lax.fori_loop(0, NCH_S, chunk, 0)
    plsc.subcore_barrier()
    pltpu.sync_copy(acc.at[pl.ds(sid * ZR, ZR)],
                    outv_hbm.at[pl.ds(c * OPAD + sid * ZR, ZR)])


MLP_TILE = 512


def _mlp_body(s_ref, p_ref, o_ref, sif_ref, oif_ref, tt_ref, pid_ref, w_ref,
              w1s_ref, w1p_ref, w1o_ref, b1a_ref, w1b_ref, b1b_ref, ptw_ref,
              newp_ref, cs_ref, co_ref, cnt_ref):
    s = s_ref[...]
    pv = p_ref[...]
    o = o_ref[...]
    h = (jnp.dot(s, w1s_ref[...], preferred_element_type=jnp.float32)
         + jnp.dot(pv, w1p_ref[...], preferred_element_type=jnp.float32)
         + jnp.dot(o, w1o_ref[...], preferred_element_type=jnp.float32)
         + b1a_ref[...])
    h = jnp.maximum(h, 0.0)
    new_t = jnp.dot(h, w1b_ref[...], preferred_element_type=jnp.float32) + b1b_ref[...]
    new_t = jnp.maximum(new_t, 0.0)

    ptp = jax.nn.sigmoid(ptw_ref[...])
    pid = pid_ref[...]
    lanes = lax.broadcasted_iota(jnp.int32, (MLP_TILE, P), 1).astype(jnp.float32)
    onehot = (lanes == pid).astype(jnp.float32)
    conf_t = jnp.sum(onehot * ptp, axis=1, keepdims=True)
    tt = tt_ref[...]
    conf = jnp.where(tt == 0.0, 1.0, conf_t)
    w = w_ref[...]
    cfw = conf * w

    newp_ref[...] = new_t[:, D:2 * D] * conf
    cs_ref[...] = new_t[:, :D] * cfw
    co_ref[...] = new_t[:, 2 * D:] * cfw

    i = pl.program_id(0)
    rows = lax.broadcasted_iota(jnp.int32, (MLP_TILE, 1), 0).astype(jnp.float32)
    batch = jnp.where(rows + i * MLP_TILE >= T, 1.0, 0.0)
    lanes160 = lax.broadcasted_iota(jnp.int32, (MLP_TILE, 2 * CROWS), 1).astype(jnp.float32)
    lanes128 = lax.broadcasted_iota(jnp.int32, (MLP_TILE, D), 1).astype(jnp.float32)

    def hist(node_f):
        hi = jnp.floor(node_f * (1.0 / 128.0))
        lo = node_f - hi * 128.0
        hirow = hi + batch * CROWS
        a = jnp.where(lanes160 == hirow, cfw, 0.0)
        bm = (lanes128 == lo).astype(jnp.float32)
        return lax.dot_general(a, bm, (((0,), (0,)), ((), ())),
                               preferred_element_type=jnp.float32)

    contrib = hist(sif_ref[...]) + hist(oif_ref[...])

    @pl.when(i == 0)
    def _():
        cnt_ref[...] = jnp.zeros_like(cnt_ref)

    cnt_ref[...] += contrib


_mlp_call = pl.pallas_call(
    _mlp_body,
    grid=(BT // MLP_TILE,),
    in_specs=[
        pl.BlockSpec((MLP_TILE, D), lambda i: (i, 0)),
        pl.BlockSpec((MLP_TILE, D), lambda i: (i, 0)),
        pl.BlockSpec((MLP_TILE, D), lambda i: (i, 0)),
        pl.BlockSpec((MLP_TILE, 1), lambda i: (i, 0)),
        pl.BlockSpec((MLP_TILE, 1), lambda i: (i, 0)),
        pl.BlockSpec((MLP_TILE, 1), lambda i: (i, 0)),
        pl.BlockSpec((MLP_TILE, 1), lambda i: (i, 0)),
        pl.BlockSpec((MLP_TILE, 1), lambda i: (i, 0)),
        pl.BlockSpec((D, H), lambda i: (0, 0)),
        pl.BlockSpec((D, H), lambda i: (0, 0)),
        pl.BlockSpec((D, H), lambda i: (0, 0)),
        pl.BlockSpec((1, H), lambda i: (0, 0)),
        pl.BlockSpec((H, W1B_OUT), lambda i: (0, 0)),
        pl.BlockSpec((1, W1B_OUT), lambda i: (0, 0)),
        pl.BlockSpec((1, P), lambda i: (0, 0)),
    ],
    out_specs=[
        pl.BlockSpec((MLP_TILE, D), lambda i: (i, 0)),
        pl.BlockSpec((MLP_TILE, D), lambda i: (i, 0)),
        pl.BlockSpec((MLP_TILE, D), lambda i: (i, 0)),
        pl.BlockSpec((2 * CROWS, D), lambda i: (0, 0)),
    ],
    out_shape=[
        jax.ShapeDtypeStruct((BT, D), jnp.float32),
        jax.ShapeDtypeStruct((BT, D), jnp.float32),
        jax.ShapeDtypeStruct((BT, D), jnp.float32),
        jax.ShapeDtypeStruct((2 * CROWS, D), jnp.float32),
    ],
    compiler_params=pltpu.CompilerParams(dimension_semantics=("arbitrary",)),
)


OUT_TILE = 2048


def _out_body(pp_ref, cnt_ref, w2a_ref, b2a_ref, w2b_ref, b2b_ref, out_ref):
    pooled = pp_ref[...]
    cnt = cnt_ref[...]
    denom = jnp.where(cnt > 0.0, cnt, 1.0)
    pn = pooled / denom
    h2 = jnp.maximum(
        jnp.dot(pn, w2a_ref[...], preferred_element_type=jnp.float32)
        + b2a_ref[...], 0.0)
    out_ref[...] = jnp.maximum(
        jnp.dot(h2, w2b_ref[...], preferred_element_type=jnp.float32)
        + b2b_ref[...], 0.0)


_out_call = pl.pallas_call(
    _out_body,
    grid=(B * OPAD // OUT_TILE,),
    in_specs=[
        pl.BlockSpec((OUT_TILE, D), lambda i: (i, 0)),
        pl.BlockSpec((OUT_TILE, 1), lambda i: (i, 0)),
        pl.BlockSpec((H, H), lambda i: (0, 0)),
        pl.BlockSpec((1, H), lambda i: (0, 0)),
        pl.BlockSpec((H, D), lambda i: (0, 0)),
        pl.BlockSpec((1, D), lambda i: (0, 0)),
    ],
    out_specs=pl.BlockSpec((OUT_TILE, D), lambda i: (i, 0)),
    out_shape=jax.ShapeDtypeStruct((B * OPAD, D), jnp.float32),
    compiler_params=pltpu.CompilerParams(dimension_semantics=("arbitrary",)),
)


def kernel(obj_vecs, pred_vecs, edges, pred_indicators, triplet_type,
           predicate_ids, W1a, b1a, W1b, b1b, W2a, b2a, W2b, b2b, ptw):
    s_idx = edges[:, :, 0]
    o_idx = edges[:, :, 1]
    boff = (jnp.arange(B, dtype=jnp.int32) * O)[:, None]
    sflat_g = (s_idx + boff).reshape(BT)
    oflat_g = (o_idx + boff).reshape(BT)
    obj_flat = obj_vecs.reshape(B * O, D)

    cur_s, cur_o = _gather_k(obj_flat, sflat_g, oflat_g)

    pred_flat = pred_vecs.reshape(BT, D)
    ttf = triplet_type.astype(jnp.float32).reshape(BT, 1)
    pidf = predicate_ids.astype(jnp.float32).reshape(BT, 1)
    wf = pred_indicators.astype(jnp.float32).reshape(BT, 1)

    sif = s_idx.astype(jnp.float32).reshape(BT, 1)
    oif = o_idx.astype(jnp.float32).reshape(BT, 1)
    new_p, cs, co, cnt = _mlp_call(
        cur_s, pred_flat, cur_o, sif, oif, ttf, pidf, wf,
        W1a[:D], W1a[D:2 * D], W1a[2 * D:], b1a.reshape(1, H),
        W1b, b1b.reshape(1, W1B_OUT), ptw.reshape(1, P))

    zeros = jnp.zeros((ZR, D), jnp.float32)
    pp = _scatter_k(s_idx.reshape(BT), o_idx.reshape(BT), cs, co, zeros)

    cnt_col = cnt.reshape(B * OPAD, 1)
    new_obj = _out_call(pp, cnt_col, W2a, b2a.reshape(1, H),
                        W2b, b2b.reshape(1, D))
    return new_obj.reshape(B, OPAD, D)[:, :O], new_p.reshape(B, T, D)
